# baseline (speedup 1.0000x reference)
import functools

import jax
import jax.numpy as jnp
from jax import lax
from jax.experimental import pallas as pl
from jax.experimental.pallas import tpu as pltpu
from jax.experimental.pallas import tpu_sc as plsc

_N = 100000
_D = 128
_B = 4096
_NBEH = 3
_INV_T = 10.0
_K = 500
_REG = (1.0, 1.0, 1.0)

_CB = 2048
_RB = 2048
_NRB = _B // _RB
_NCB = 49
_M = _NCB * 128
_PAD_NEG = -3.0

_SC_WORKERS = 32
_BPW = _B // _SC_WORKERS


def _normalize_rows(x):
    ssq = jnp.sum(x * x, axis=1, keepdims=True)
    return x * lax.rsqrt(jnp.maximum(ssq, 1e-24))


def _select_beh(g3, a):
    return jnp.where(a == 0, g3[:, 0, :],
                     jnp.where(a == 1, g3[:, 1, :], g3[:, 2, :]))


@functools.cache
def _gather_kernel():
    @functools.partial(
        pl.kernel,
        mesh=plsc.VectorSubcoreMesh(core_axis_name="c", subcore_axis_name="s"),
        out_type=jax.ShapeDtypeStruct((_B, _NBEH, _D), jnp.float32),
        scratch_types=[
            pltpu.VMEM((_BPW,), jnp.int32),
            pltpu.VMEM((_BPW, _NBEH, _D), jnp.float32),
            pltpu.SemaphoreType.DMA,
        ],
    )
    def gather(table_hbm, idx_hbm, out_hbm, idx_v, rows_v, sem):
        wid = lax.axis_index("s") * 2 + lax.axis_index("c")
        base = wid * _BPW
        pltpu.sync_copy(idx_hbm.at[pl.ds(base, _BPW)], idx_v)
        pltpu.async_copy(table_hbm.at[idx_v], rows_v, sem).wait()
        pltpu.sync_copy(rows_v, out_hbm.at[pl.ds(base, _BPW)])

    return gather


def _gather_rows(emb, idx):
    return _gather_kernel()(emb, idx)


def _prep_body(aux_sref, emb_ref, out_ref):
    a = aux_sref[0]
    aux = _select_beh(emb_ref[...], a)
    out_ref[...] = _normalize_rows(aux).astype(jnp.bfloat16)


def _prep_auxn(emb, aux_idx):
    return pl.pallas_call(
        _prep_body,
        grid_spec=pltpu.PrefetchScalarGridSpec(
            num_scalar_prefetch=1,
            grid=(_NCB,),
            in_specs=[pl.BlockSpec((_CB, _NBEH, _D), lambda j, a: (j, 0, 0))],
            out_specs=pl.BlockSpec((_CB, _D), lambda j, a: (j, 0)),
        ),
        out_shape=jax.ShapeDtypeStruct((_NCB * _CB, _D), jnp.bfloat16),
        compiler_params=pltpu.CompilerParams(
            dimension_semantics=("parallel",)),
    )(aux_idx, emb)


def _scores_block(tgtn_f32, auxn_bf16):
    return lax.dot_general(tgtn_f32.astype(jnp.bfloat16), auxn_bf16,
                           (((1,), (1,)), ((), ())),
                           preferred_element_type=jnp.float32)


def _strided_max(s):
    m = s[:, 0:128]
    for t in range(1, _CB // 128):
        m = jnp.maximum(m, s[:, t * 128:(t + 1) * 128])
    return m


def _maxima_body(gath_ref, auxn_ref, out_ref, tgtn_ref):
    j = pl.program_id(1)

    @pl.when(j == 0)
    def _():
        tgtn_ref[...] = _normalize_rows(gath_ref[:, _NBEH - 1, :])

    s = _scores_block(tgtn_ref[...], auxn_ref[...])

    @pl.when(j < _NCB - 1)
    def _():
        out_ref[...] = _strided_max(s).astype(jnp.bfloat16)

    @pl.when(j == _NCB - 1)
    def _():
        col = j * _CB + lax.broadcasted_iota(jnp.int32, (1, _CB), 1)
        sm = jnp.where(col < _N, s, _PAD_NEG)
        out_ref[...] = _strided_max(sm).astype(jnp.bfloat16)


def _chunk_maxima(gath, auxn):
    return pl.pallas_call(
        _maxima_body,
        grid=(_NRB, _NCB),
        in_specs=[
            pl.BlockSpec((_RB, _NBEH, _D), lambda rb, j: (rb, 0, 0)),
            pl.BlockSpec((_CB, _D), lambda rb, j: (j, 0)),
        ],
        out_specs=pl.BlockSpec((_RB, 128), lambda rb, j: (rb, j)),
        out_shape=jax.ShapeDtypeStruct((_B, _M), jnp.bfloat16),
        scratch_shapes=[pltpu.VMEM((_RB, _D), jnp.float32)],
        compiler_params=pltpu.CompilerParams(
            dimension_semantics=("parallel", "arbitrary")),
    )(gath, auxn)


def _cutoff_body(cm_ref, x_ref):
    c = cm_ref[...]
    rows = c.shape[0]

    def it(_, lh):
        lo, hi = lh
        mid = 0.5 * (lo + hi)
        cmp = (c >= mid.astype(jnp.bfloat16)).astype(jnp.float32)
        cnt = jnp.sum(cmp, axis=1, keepdims=True)
        ge = cnt >= float(_K)
        return jnp.where(ge, mid, lo), jnp.where(ge, hi, mid)

    lo0 = jnp.full((rows, 1), -1.001, jnp.float32)
    hi0 = jnp.full((rows, 1), 1.001, jnp.float32)
    lo, _ = lax.fori_loop(0, 24, it, (lo0, hi0))
    x_ref[...] = lo


def _cutoff(cm):
    rb = 512
    return pl.pallas_call(
        _cutoff_body,
        grid=(_B // rb,),
        in_specs=[pl.BlockSpec((rb, _M), lambda i: (i, 0))],
        out_specs=pl.BlockSpec((rb, 1), lambda i: (i, 0)),
        out_shape=jax.ShapeDtypeStruct((_B, 1), jnp.float32),
        compiler_params=pltpu.CompilerParams(
            dimension_semantics=("parallel",)),
    )(cm)


def _loss_body(aux_sref, gath_ref, auxn_ref, x_ref, out_ref,
               tgtn_ref, acc_ref):
    j = pl.program_id(1)

    @pl.when(j == 0)
    def _():
        tgtn_ref[...] = _normalize_rows(gath_ref[:, _NBEH - 1, :])

    s = _scores_block(tgtn_ref[...], auxn_ref[...])
    kept = s >= x_ref[...]

    @pl.when(j < _NCB - 1)
    def _():
        contrib = jnp.where(kept, jnp.exp(s * _INV_T), 1.0)
        psum = jnp.sum(contrib, axis=1, keepdims=True)
        acc_ref[...] = jnp.where(j == 0, psum, acc_ref[...] + psum)

    @pl.when(j == _NCB - 1)
    def _():
        col = j * _CB + lax.broadcasted_iota(jnp.int32, (1, _CB), 1)
        valid = col < _N
        contrib = jnp.where(jnp.logical_and(valid, kept), jnp.exp(s * _INV_T),
                            jnp.where(valid, 1.0, 0.0))
        ttl = acc_ref[...] + jnp.sum(contrib, axis=1, keepdims=True)
        a = aux_sref[0]
        gn = _normalize_rows(_select_beh(gath_ref[...], a))
        pos = jnp.sum(tgtn_ref[...] * gn, axis=1, keepdims=True)
        v = jnp.sum(jnp.log(ttl) - pos * _INV_T)
        out_ref[...] = jnp.zeros((1, 8, 128), jnp.float32) + v


def _side_loss_scalar(gath, auxn, x, aux_idx):
    return pl.pallas_call(
        _loss_body,
        grid_spec=pltpu.PrefetchScalarGridSpec(
            num_scalar_prefetch=1,
            grid=(_NRB, _NCB),
            in_specs=[
                pl.BlockSpec((_RB, _NBEH, _D), lambda rb, j, a: (rb, 0, 0)),
                pl.BlockSpec((_CB, _D), lambda rb, j, a: (j, 0)),
                pl.BlockSpec((_RB, 1), lambda rb, j, a: (rb, 0)),
            ],
            out_specs=pl.BlockSpec((1, 8, 128), lambda rb, j, a: (rb, 0, 0)),
            scratch_shapes=[
                pltpu.VMEM((_RB, _D), jnp.float32),
                pltpu.VMEM((_RB, 1), jnp.float32),
            ],
        ),
        out_shape=jax.ShapeDtypeStruct((_NRB, 8, 128), jnp.float32),
        compiler_params=pltpu.CompilerParams(
            dimension_semantics=("parallel", "arbitrary")),
    )(aux_idx, gath, auxn, x)


def _one_side(idx, emb, aux_idx):
    gath = _gather_rows(emb, idx)
    auxn = _prep_auxn(emb, aux_idx)
    cm = _chunk_maxima(gath, auxn)
    x = _cutoff(cm)
    return jnp.sum(_side_loss_scalar(gath, auxn, x, aux_idx)[:, 0, 0])


def kernel(input_u_list, input_i_list, ua_embeddings, ia_embeddings, aux_beh):
    aux_idx = jnp.asarray(aux_beh, jnp.int32).reshape(1)
    loss_u = _one_side(input_u_list.astype(jnp.int32), ua_embeddings, aux_idx)
    loss_i = _one_side(input_i_list.astype(jnp.int32), ia_embeddings, aux_idx)
    return (loss_u + loss_i) * jnp.asarray(_REG, jnp.float32)[aux_idx[0]]
